# baseline (speedup 1.0000x reference)
import functools

import jax
import jax.numpy as jnp
from jax import lax
from jax.experimental import pallas as pl
from jax.experimental.pallas import tpu as pltpu
from jax.experimental.pallas import tpu_sc as plsc

_B, _N, _D = 128, 8192, 16
_RSC = 32
_RTC = _B - _RSC
_NW = 32
_RPW = _RSC // _NW
_CW = 2048
_NCH = _N // _CW
_NG = _RPW * _NCH
_UNR = 2
_TCB = 8


def _sc_body(x_hbm, o_hbm, buf0, buf1, win, wout, sem0, sem1):
    wid = lax.axis_index("s") * 2 + lax.axis_index("c")
    iota = lax.iota(jnp.int32, 16)

    bufs = (buf0, buf1)
    sems = (sem0, sem1)

    def start(g):
        row = wid * _RPW + (g // _NCH)
        cap0 = (g % _NCH) * _CW
        return pltpu.async_copy(
            x_hbm.at[row, :, pl.ds(cap0, _CW)], bufs[g % 2], sems[g % 2])

    cur = start(0)
    bv = bi = None
    for g in range(_NG):
        nxt = start(g + 1) if g + 1 < _NG else None
        cur.wait()
        buf = bufs[g % 2]
        c = g % _NCH
        if c == 0:
            bv = [jnp.full((16,), -1.0, jnp.float32) for _ in range(_UNR)]
            bi = [jnp.zeros((16,), jnp.int32) for _ in range(_UNR)]
        cbase = c * _CW

        def grp(q, carry, buf=buf, cbase=cbase):
            sbv = list(carry[: _UNR])
            sbi = list(carry[_UNR:])
            for m in range(_UNR):
                vs = [buf[k, pl.ds(q * (16 * _UNR) + m * 16, 16)]
                      for k in range(_D)]
                sq = [v * v for v in vs]
                while len(sq) > 1:
                    sq = [sq[i] + sq[i + 1] for i in range(0, len(sq), 2)]
                acc = sq[0]
                idxv = iota + (cbase + m * 16) + q * (16 * _UNR)
                m_upd = acc > sbv[m]
                sbv[m] = jnp.where(m_upd, acc, sbv[m])
                sbi[m] = jnp.where(m_upd, idxv, sbi[m])
            return tuple(sbv) + tuple(sbi)

        res = lax.fori_loop(0, _CW // (16 * _UNR), grp,
                            tuple(bv) + tuple(bi))
        bv = list(res[: _UNR])
        bi = list(res[_UNR:])

        if c == _NCH - 1:
            row = wid * _RPW + (g // _NCH)
            mv, mi = bv[0], bi[0]
            for u in range(1, _UNR):
                take = (bv[u] > mv) | ((bv[u] == mv) & (bi[u] < mi))
                mv = jnp.where(take, bv[u], mv)
                mi = jnp.where(take, bi[u], mi)
            mx = jnp.max(mv)
            cand = jnp.where(mv == mx, mi, jnp.int32(_N))
            j = jnp.min(cand)
            jt = pl.multiple_of((j >> 7) << 7, 128)
            pltpu.sync_copy(x_hbm.at[row, :, pl.ds(jt, 128)], win)
            jm = jnp.full((16,), 0, jnp.int32) + (j - jt)
            wv = plsc.load_gather(win, [iota, jm])
            wout[0, pl.ds(0, _D)] = wv
            pltpu.sync_copy(wout, o_hbm.at[pl.ds(row, 1), :])
        cur = nxt


_sc_part = functools.partial(
    pl.kernel,
    out_type=jax.ShapeDtypeStruct((_RSC, _D), jnp.float32),
    mesh=plsc.VectorSubcoreMesh(core_axis_name="c", subcore_axis_name="s"),
    compiler_params=pltpu.CompilerParams(
        needs_layout_passes=False, use_tc_tiling_on_sc=True),
    scratch_types=[
        pltpu.VMEM((_D, _CW), jnp.float32),
        pltpu.VMEM((_D, _CW), jnp.float32),
        pltpu.VMEM((_D, 128), jnp.float32),
        pltpu.VMEM((1, _D), jnp.float32),
        pltpu.SemaphoreType.DMA,
        pltpu.SemaphoreType.DMA,
    ],
)(_sc_body)


def _tc_body(x_ref, o_ref):
    x = x_ref[...]
    x2d = x.reshape(_TCB * _D, _N)
    sq = x2d * x2d
    sel = (lax.broadcasted_iota(jnp.int32, (_TCB, _TCB * _D), 1) // _D
           == lax.broadcasted_iota(jnp.int32, (_TCB, _TCB * _D), 0)
           ).astype(jnp.float32)
    n2 = jax.lax.dot_general(sel, sq, (((1,), (0,)), ((), ())),
                             precision=lax.Precision.HIGHEST,
                             preferred_element_type=jnp.float32)
    m = jnp.max(n2, axis=1, keepdims=True)
    iota2 = lax.broadcasted_iota(jnp.int32, (_TCB, _N), 1)
    cand = jnp.where(n2 == m, iota2, jnp.int32(_N))
    j = jnp.min(cand, axis=1, keepdims=True)
    oh = (iota2 == j).astype(jnp.float32)
    w = jax.lax.dot_general(x2d, oh, (((1,), (1,)), ((), ())),
                            precision=lax.Precision.HIGHEST,
                            preferred_element_type=jnp.float32)
    wr = w.reshape(_TCB, _D, _TCB)
    picked = (lax.broadcasted_iota(jnp.int32, (_TCB, _D, _TCB), 0)
              == lax.broadcasted_iota(jnp.int32, (_TCB, _D, _TCB), 2))
    o_ref[...] = jnp.sum(jnp.where(picked, wr, 0.0), axis=2)


_tc_part = pl.pallas_call(
    _tc_body,
    out_shape=jax.ShapeDtypeStruct((_RTC, _D), jnp.float32),
    grid=(_RTC // _TCB,),
    in_specs=[pl.BlockSpec((_TCB, _D, _N),
                           lambda i: (_RSC // _TCB + i, 0, 0))],
    out_specs=pl.BlockSpec((_TCB, _D), lambda i: (i, 0)),
    compiler_params=pltpu.CompilerParams(
        dimension_semantics=("arbitrary",)),
)


@jax.jit
def kernel(inputs):
    xt = jnp.transpose(inputs, (0, 2, 1))
    sc_out = _sc_part(xt)
    tc_out = _tc_part(xt)
    return jnp.concatenate([sc_out, tc_out], axis=0)


# speedup vs baseline: 2.0636x; 2.0636x over previous
import functools

import jax
import jax.numpy as jnp
from jax import lax
from jax.experimental import pallas as pl
from jax.experimental.pallas import tpu as pltpu
from jax.experimental.pallas import tpu_sc as plsc

_B, _N, _D = 128, 8192, 16
_RSC = 64
_RTC = _B - _RSC
_NW = 32
_RPW = _RSC // _NW
_CW = 2048
_NCH = _N // _CW
_NG = _RPW * _NCH
_UNR = 2
_TCB = 16


def _sc_body(x_hbm, o_hbm, buf0, buf1, win, wout, sem0, sem1):
    wid = lax.axis_index("s") * 2 + lax.axis_index("c")
    iota = lax.iota(jnp.int32, 16)

    bufs = (buf0, buf1)
    sems = (sem0, sem1)

    def start(g):
        row = wid * _RPW + (g // _NCH)
        cap0 = (g % _NCH) * _CW
        return pltpu.async_copy(
            x_hbm.at[row, :, pl.ds(cap0, _CW)], bufs[g % 2], sems[g % 2])

    cur = start(0)
    bv = bi = None
    for g in range(_NG):
        nxt = start(g + 1) if g + 1 < _NG else None
        cur.wait()
        buf = bufs[g % 2]
        c = g % _NCH
        if c == 0:
            bv = [jnp.full((16,), -1.0, jnp.float32) for _ in range(_UNR)]
            bi = [jnp.zeros((16,), jnp.int32) for _ in range(_UNR)]
        cbase = c * _CW

        def grp(q, carry, buf=buf, cbase=cbase):
            sbv = list(carry[: _UNR])
            sbi = list(carry[_UNR:])
            for m in range(_UNR):
                vs = [buf[k, pl.ds(q * (16 * _UNR) + m * 16, 16)]
                      for k in range(_D)]
                sq = [v * v for v in vs]
                while len(sq) > 1:
                    sq = [sq[i] + sq[i + 1] for i in range(0, len(sq), 2)]
                acc = sq[0]
                idxv = iota + (cbase + m * 16) + q * (16 * _UNR)
                m_upd = acc > sbv[m]
                sbv[m] = jnp.where(m_upd, acc, sbv[m])
                sbi[m] = jnp.where(m_upd, idxv, sbi[m])
            return tuple(sbv) + tuple(sbi)

        res = lax.fori_loop(0, _CW // (16 * _UNR), grp,
                            tuple(bv) + tuple(bi))
        bv = list(res[: _UNR])
        bi = list(res[_UNR:])

        if c == _NCH - 1:
            row = wid * _RPW + (g // _NCH)
            mv, mi = bv[0], bi[0]
            for u in range(1, _UNR):
                take = (bv[u] > mv) | ((bv[u] == mv) & (bi[u] < mi))
                mv = jnp.where(take, bv[u], mv)
                mi = jnp.where(take, bi[u], mi)
            mx = jnp.max(mv)
            cand = jnp.where(mv == mx, mi, jnp.int32(_N))
            j = jnp.min(cand)
            jt = pl.multiple_of((j >> 7) << 7, 128)
            pltpu.sync_copy(x_hbm.at[row, :, pl.ds(jt, 128)], win)
            jm = jnp.full((16,), 0, jnp.int32) + (j - jt)
            wv = plsc.load_gather(win, [iota, jm])
            wout[0, pl.ds(0, _D)] = wv
            pltpu.sync_copy(wout, o_hbm.at[pl.ds(row, 1), :])
        cur = nxt


_sc_part = functools.partial(
    pl.kernel,
    out_type=jax.ShapeDtypeStruct((_RSC, _D), jnp.float32),
    mesh=plsc.VectorSubcoreMesh(core_axis_name="c", subcore_axis_name="s"),
    compiler_params=pltpu.CompilerParams(
        needs_layout_passes=False, use_tc_tiling_on_sc=True),
    scratch_types=[
        pltpu.VMEM((_D, _CW), jnp.float32),
        pltpu.VMEM((_D, _CW), jnp.float32),
        pltpu.VMEM((_D, 128), jnp.float32),
        pltpu.VMEM((1, _D), jnp.float32),
        pltpu.SemaphoreType.DMA,
        pltpu.SemaphoreType.DMA,
    ],
)(_sc_body)


def _tc_body(x_ref, o_ref):
    x = x_ref[...]
    n2 = jnp.sum(x * x, axis=1)
    m = jnp.max(n2, axis=1, keepdims=True)
    iota2 = lax.broadcasted_iota(jnp.int32, (_TCB, _N), 1)
    cand = jnp.where(n2 == m, iota2, jnp.int32(_N))
    j = jnp.min(cand, axis=1)
    oh = (iota2 == j[:, None]).astype(jnp.float32)
    o_ref[...] = jnp.einsum('bkn,bn->bk', x, oh,
                            preferred_element_type=jnp.float32)


_tc_part = pl.pallas_call(
    _tc_body,
    out_shape=jax.ShapeDtypeStruct((_RTC, _D), jnp.float32),
    grid=(_RTC // _TCB,),
    in_specs=[pl.BlockSpec((_TCB, _D, _N),
                           lambda i: (_RSC // _TCB + i, 0, 0))],
    out_specs=pl.BlockSpec((_TCB, _D), lambda i: (i, 0)),
    compiler_params=pltpu.CompilerParams(
        dimension_semantics=("arbitrary",)),
)


@jax.jit
def kernel(inputs):
    xt = jnp.transpose(inputs, (0, 2, 1))
    sc_out = _sc_part(xt)
    tc_out = _tc_part(xt)
    return jnp.concatenate([sc_out, tc_out], axis=0)


# speedup vs baseline: 2.0882x; 1.0119x over previous
import functools

import jax
import jax.numpy as jnp
from jax import lax
from jax.experimental import pallas as pl
from jax.experimental.pallas import tpu as pltpu
from jax.experimental.pallas import tpu_sc as plsc

_B, _N, _D = 128, 8192, 16
_RSC = 64
_RTC = _B - _RSC
_NW = 32
_RPW = _RSC // _NW
_CW = 2048
_NCH = _N // _CW
_NG = _RPW * _NCH
_UNR = 2
_TCB = 16


def _sc_body(x_hbm, o_hbm, buf0, buf1, win, wout, sem0, sem1):
    wid = lax.axis_index("s") * 2 + lax.axis_index("c")
    iota = lax.iota(jnp.int32, 16)

    bufs = (buf0, buf1)
    sems = (sem0, sem1)

    def start(g):
        row = wid * _RPW + (g // _NCH)
        cap0 = (g % _NCH) * _CW
        return pltpu.async_copy(
            x_hbm.at[row, :, pl.ds(cap0, _CW)], bufs[g % 2], sems[g % 2])

    def chunk_src(g):
        row = wid * _RPW + lax.div(g, _NCH)
        cap0 = lax.rem(g, _NCH) * _CW
        return x_hbm.at[row, :, pl.ds(cap0, _CW)]

    start(0)
    start(1)

    def scan_chunk(g, buf, carry):
        def grp(q, carry):
            sbv = list(carry[: _UNR])
            sbi = list(carry[_UNR: 2 * _UNR])
            cbase = carry[2 * _UNR]
            for m in range(_UNR):
                vs = [buf[k, pl.ds(q * (16 * _UNR) + m * 16, 16)]
                      for k in range(_D)]
                sq = [v * v for v in vs]
                while len(sq) > 1:
                    sq = [sq[i] + sq[i + 1] for i in range(0, len(sq), 2)]
                acc = sq[0]
                idxv = iota + (cbase + m * 16) + q * (16 * _UNR)
                m_upd = acc > sbv[m]
                sbv[m] = jnp.where(m_upd, acc, sbv[m])
                sbi[m] = jnp.where(m_upd, idxv, sbi[m])
            return tuple(sbv) + tuple(sbi) + (cbase,)

        cbase = lax.rem(g, _NCH) * _CW
        res = lax.fori_loop(0, _CW // (16 * _UNR), grp, carry + (cbase,))
        return res[: 2 * _UNR]

    def finish_row(g, carry):
        bv = carry[: _UNR]
        bi = carry[_UNR:]
        row = wid * _RPW + lax.div(g, _NCH)
        mv, mi = bv[0], bi[0]
        for u in range(1, _UNR):
            take = (bv[u] > mv) | ((bv[u] == mv) & (bi[u] < mi))
            mv = jnp.where(take, bv[u], mv)
            mi = jnp.where(take, bi[u], mi)
        mx = jnp.max(mv)
        cand = jnp.where(mv == mx, mi, jnp.int32(_N))
        j = jnp.min(cand)
        jt = pl.multiple_of((j >> 7) << 7, 128)
        pltpu.sync_copy(x_hbm.at[row, :, pl.ds(jt, 128)], win)
        jm = jnp.full((16,), 0, jnp.int32) + (j - jt)
        wv = plsc.load_gather(win, [iota, jm])
        wout[0, pl.ds(0, _D)] = wv
        pltpu.sync_copy(wout, o_hbm.at[pl.ds(row, 1), :])

    def ring(ig, carry):
        g0 = ig * 2
        g1 = g0 + 1
        fresh = lax.rem(g0, _NCH) == 0
        carry = tuple(
            jnp.where(fresh, jnp.full((16,), -1.0, jnp.float32), c)
            for c in carry[: _UNR]) + tuple(
            jnp.where(fresh, jnp.zeros((16,), jnp.int32), c)
            for c in carry[_UNR:])
        pltpu.make_async_copy(chunk_src(g0), buf0, sem0).wait()
        carry = scan_chunk(g0, buf0, carry)

        @pl.when(g0 + 2 < _NG)
        def _():
            pltpu.async_copy(chunk_src(g0 + 2), buf0, sem0)

        pltpu.make_async_copy(chunk_src(g1), buf1, sem1).wait()
        carry = scan_chunk(g1, buf1, carry)

        @pl.when(g1 + 2 < _NG)
        def _():
            pltpu.async_copy(chunk_src(g1 + 2), buf1, sem1)

        @pl.when(lax.rem(g1, _NCH) == _NCH - 1)
        def _():
            finish_row(g1, carry)

        return carry

    init = tuple(jnp.full((16,), -1.0, jnp.float32) for _ in range(_UNR)) \
        + tuple(jnp.zeros((16,), jnp.int32) for _ in range(_UNR))
    lax.fori_loop(0, _NG // 2, ring, init)


_sc_part = functools.partial(
    pl.kernel,
    out_type=jax.ShapeDtypeStruct((_RSC, _D), jnp.float32),
    mesh=plsc.VectorSubcoreMesh(core_axis_name="c", subcore_axis_name="s"),
    compiler_params=pltpu.CompilerParams(
        needs_layout_passes=False, use_tc_tiling_on_sc=True),
    scratch_types=[
        pltpu.VMEM((_D, _CW), jnp.float32),
        pltpu.VMEM((_D, _CW), jnp.float32),
        pltpu.VMEM((_D, 128), jnp.float32),
        pltpu.VMEM((1, _D), jnp.float32),
        pltpu.SemaphoreType.DMA,
        pltpu.SemaphoreType.DMA,
    ],
)(_sc_body)


def _tc_body(x_ref, o_ref):
    x = x_ref[...]
    n2 = jnp.sum(x * x, axis=1)
    m = jnp.max(n2, axis=1, keepdims=True)
    iota2 = lax.broadcasted_iota(jnp.int32, (_TCB, _N), 1)
    cand = jnp.where(n2 == m, iota2, jnp.int32(_N))
    j = jnp.min(cand, axis=1)
    oh = (iota2 == j[:, None]).astype(jnp.float32)
    o_ref[...] = jnp.einsum('bkn,bn->bk', x, oh,
                            preferred_element_type=jnp.float32)


_tc_part = pl.pallas_call(
    _tc_body,
    out_shape=jax.ShapeDtypeStruct((_RTC, _D), jnp.float32),
    grid=(_RTC // _TCB,),
    in_specs=[pl.BlockSpec((_TCB, _D, _N),
                           lambda i: (_RSC // _TCB + i, 0, 0))],
    out_specs=pl.BlockSpec((_TCB, _D), lambda i: (i, 0)),
    compiler_params=pltpu.CompilerParams(
        dimension_semantics=("arbitrary",)),
)


@jax.jit
def kernel(inputs):
    xt = jnp.transpose(inputs, (0, 2, 1))
    sc_out = _sc_part(xt)
    tc_out = _tc_part(xt)
    return jnp.concatenate([sc_out, tc_out], axis=0)
